# baseline (speedup 1.0000x reference)
import functools

import jax
import jax.numpy as jnp
from jax import lax
from jax.experimental import pallas as pl
from jax.experimental.pallas import tpu as pltpu
from jax.experimental.pallas import tpu_sc as plsc

N = 10000
E = 320000
D = 128

NC = 2
NS = 16
NW = NC * NS

K = 128
KH = K // 2
STEPS = 80
HALF = STEPS // 2
E_PAD = NW * STEPS * K
DUMMY = N

DEG_ACC = 10240
DEG_SLAB = DEG_ACC // NS
MSG_ACC = 10112
MSG_SLAB = MSG_ACC // NS


def _mesh():
    return plsc.VectorSubcoreMesh(
        core_axis_name="c", subcore_axis_name="s", num_cores=NC, num_subcores=NS)


def _deg_body(dst_hbm, zero_hbm, out_hbm, dstv, onesv, dacc, sem):
    c = lax.axis_index("c")
    s = lax.axis_index("s")
    w = c * NS + s
    pltpu.sync_copy(dst_hbm.at[w], dstv)
    for i in range(K // 16):
        onesv[pl.ds(i * 16, 16)] = jnp.ones((16,), jnp.float32)
    pltpu.sync_copy(zero_hbm.at[pl.ds(s * DEG_SLAB, DEG_SLAB)],
                    dacc.at[pl.ds(s * DEG_SLAB, DEG_SLAB)])
    plsc.subcore_barrier()

    def step(j, carry):
        pltpu.sync_copy(onesv, dacc.at[dstv.at[j]], add=True)
        return carry

    lax.fori_loop(0, STEPS, step, 0)
    plsc.subcore_barrier()
    pltpu.sync_copy(dacc.at[pl.ds(s * DEG_SLAB, DEG_SLAB)],
                    out_hbm.at[pl.ds(c * DEG_ACC + s * DEG_SLAB, DEG_SLAB)])


def _deg_call(dstp, zeros1):
    return pl.kernel(
        _deg_body,
        out_type=jax.ShapeDtypeStruct((NC * DEG_ACC,), jnp.float32),
        mesh=_mesh(),
        scratch_types=[
            pltpu.VMEM((STEPS, K), jnp.int32),
            pltpu.VMEM((K,), jnp.float32),
            pltpu.VMEM_SHARED((DEG_ACC,), jnp.float32),
            pltpu.SemaphoreType.DMA,
        ],
    )(dstp, zeros1)


OUTER = STEPS // 2


def _msg_body(g_hbm, src_hbm, dst_hbm, zero_hbm, out_hbm,
              srcv, dstv, bufA, bufB, sacc, gsem):
    c = lax.axis_index("c")
    s = lax.axis_index("s")
    w = c * NS + s
    pltpu.sync_copy(src_hbm.at[w], srcv)
    pltpu.sync_copy(dst_hbm.at[w, pl.ds(0, HALF)], dstv)
    pltpu.sync_copy(zero_hbm.at[pl.ds(s * MSG_SLAB, MSG_SLAB)],
                    sacc.at[pl.ds(s * MSG_SLAB, MSG_SLAB)])
    plsc.subcore_barrier()

    def fire_g(j, buf):
        pltpu.async_copy(g_hbm.at[srcv.at[j, pl.ds(0, KH)]],
                         buf.at[pl.ds(0, KH)], gsem)
        pltpu.async_copy(g_hbm.at[srcv.at[j, pl.ds(KH, KH)]],
                         buf.at[pl.ds(KH, KH)], gsem)

    def scat(j, buf):
        pltpu.sync_copy(buf, sacc.at[dstv.at[lax.rem(j, HALF)]], add=True)

    def drain():
        pltpu.make_async_copy(g_hbm.at[pl.ds(0, KH)],
                              bufA.at[pl.ds(0, KH)], gsem).wait()
        pltpu.make_async_copy(g_hbm.at[pl.ds(0, KH)],
                              bufA.at[pl.ds(0, KH)], gsem).wait()

    fire_g(0, bufA)

    def outer(i, carry):
        a = 2 * i
        b_ = 2 * i + 1

        @pl.when(a == HALF)
        def _():
            pltpu.sync_copy(dst_hbm.at[w, pl.ds(HALF, HALF)], dstv)

        drain()
        fire_g(b_, bufB)
        scat(a, bufA)
        drain()

        @pl.when(i < OUTER - 1)
        def _():
            fire_g(a + 2, bufA)

        scat(b_, bufB)
        return carry

    lax.fori_loop(0, OUTER, outer, 0)
    plsc.subcore_barrier()
    pltpu.sync_copy(sacc.at[pl.ds(s * MSG_SLAB, MSG_SLAB)],
                    out_hbm.at[c, pl.ds(s * MSG_SLAB, MSG_SLAB)])


def _msg_call(g, srcp, dstp, zeros2):
    return pl.kernel(
        _msg_body,
        out_type=jax.ShapeDtypeStruct((NC, MSG_ACC, D), jnp.float32),
        mesh=_mesh(),
        scratch_types=[
            pltpu.VMEM((STEPS, K), jnp.int32),
            pltpu.VMEM((HALF, K), jnp.int32),
            pltpu.VMEM((K, D), jnp.float32),
            pltpu.VMEM((K, D), jnp.float32),
            pltpu.VMEM_SHARED((MSG_ACC, D), jnp.float32),
            pltpu.SemaphoreType.DMA,
        ],
    )(g, srcp, dstp, zeros2)


_MM_BM = 2000


def _mm_body(x_ref, w_ref, h_ref):
    h_ref[...] = jnp.dot(x_ref[...], w_ref[...],
                         preferred_element_type=jnp.float32)


def _mm_call(x, W):
    return pl.pallas_call(
        _mm_body,
        grid=(N // _MM_BM,),
        in_specs=[
            pl.BlockSpec((_MM_BM, D), lambda i: (i, 0)),
            pl.BlockSpec((D, D), lambda i: (0, 0)),
        ],
        out_specs=pl.BlockSpec((_MM_BM, D), lambda i: (i, 0)),
        out_shape=jax.ShapeDtypeStruct((N, D), jnp.float32),
    )(x, W)


def _scale_body(deg_ref, h_ref, g_ref):
    d = deg_ref[0] + deg_ref[1] + 1.0
    dinv = lax.rsqrt(d)
    g_ref[...] = h_ref[...] * dinv


def _scale_call(deg_col, h):
    bm = 2000
    return pl.pallas_call(
        _scale_body,
        grid=(N // bm,),
        in_specs=[
            pl.BlockSpec((NC, bm, 1), lambda i: (0, i, 0)),
            pl.BlockSpec((bm, D), lambda i: (i, 0)),
        ],
        out_specs=pl.BlockSpec((bm, D), lambda i: (i, 0)),
        out_shape=jax.ShapeDtypeStruct((N, D), jnp.float32),
    )(deg_col, h)


def _out_body(deg_ref, s_ref, g_ref, b_ref, o_ref):
    d = deg_ref[0] + deg_ref[1] + 1.0
    dinv = lax.rsqrt(d)
    o_ref[...] = dinv * (s_ref[0] + s_ref[1] + g_ref[...]) + b_ref[...]


def _out_call(deg_col, s2, g, b2):
    bm = 2000
    return pl.pallas_call(
        _out_body,
        grid=(N // bm,),
        in_specs=[
            pl.BlockSpec((NC, bm, 1), lambda i: (0, i, 0)),
            pl.BlockSpec((NC, bm, D), lambda i: (0, i, 0)),
            pl.BlockSpec((bm, D), lambda i: (i, 0)),
            pl.BlockSpec((1, D), lambda i: (0, 0)),
        ],
        out_specs=pl.BlockSpec((bm, D), lambda i: (i, 0)),
        out_shape=jax.ShapeDtypeStruct((N, D), jnp.float32),
    )(deg_col, s2, g, b2)


def kernel(x, edge_index, edge_attr, W, b):
    src = edge_index[0].astype(jnp.int32)
    dst = edge_index[1].astype(jnp.int32)
    pad = E_PAD - E
    srcp = jnp.concatenate(
        [src, jnp.zeros((pad,), jnp.int32)]).reshape(NW, STEPS, K)
    dstp = jnp.concatenate(
        [dst, jnp.full((pad,), DUMMY, jnp.int32)]).reshape(NW, STEPS, K)
    zeros1 = jnp.zeros((DEG_ACC,), jnp.float32)
    zeros2 = jnp.zeros((MSG_ACC, D), jnp.float32)

    deg2 = _deg_call(dstp, zeros1).reshape(NC, DEG_ACC)
    h = _mm_call(x, W)
    deg_col = deg2[:, :N].reshape(NC, N, 1)
    g = _scale_call(deg_col, h)
    s2 = _msg_call(g, srcp, dstp, zeros2)[:, :N, :]
    return _out_call(deg_col, s2, g, b.reshape(1, D))


# speedup vs baseline: 1.2118x; 1.2118x over previous
import functools

import jax
import jax.numpy as jnp
from jax import lax
from jax.experimental import pallas as pl
from jax.experimental.pallas import tpu as pltpu
from jax.experimental.pallas import tpu_sc as plsc

N = 10000
E = 320000
D = 128

NC = 2
NS = 16
NW = NC * NS

K = 128
KH = K // 2
STEPS = 80
HALF = STEPS // 2
E_PAD = NW * STEPS * K
DUMMY = N

DEG_ACC = 10240
DEG_SLAB = DEG_ACC // NS
MSG_ACC = 10112
MSG_SLAB = MSG_ACC // NS


def _mesh():
    return plsc.VectorSubcoreMesh(
        core_axis_name="c", subcore_axis_name="s", num_cores=NC, num_subcores=NS)


def _deg_body(dst_hbm, zero_hbm, out_hbm, dstv, onesv, dacc, sem):
    c = lax.axis_index("c")
    s = lax.axis_index("s")
    w = c * NS + s
    pltpu.sync_copy(dst_hbm.at[w], dstv)
    for i in range(K // 16):
        onesv[pl.ds(i * 16, 16)] = jnp.ones((16,), jnp.float32)
    pltpu.sync_copy(zero_hbm.at[pl.ds(s * DEG_SLAB, DEG_SLAB)],
                    dacc.at[pl.ds(s * DEG_SLAB, DEG_SLAB)])
    plsc.subcore_barrier()

    def step(j, carry):
        pltpu.sync_copy(onesv, dacc.at[dstv.at[j]], add=True)
        return carry

    lax.fori_loop(0, STEPS, step, 0)
    plsc.subcore_barrier()
    pltpu.sync_copy(dacc.at[pl.ds(s * DEG_SLAB, DEG_SLAB)],
                    out_hbm.at[pl.ds(c * DEG_ACC + s * DEG_SLAB, DEG_SLAB)])


def _deg_call(dstp, zeros1):
    return pl.kernel(
        _deg_body,
        out_type=jax.ShapeDtypeStruct((NC * DEG_ACC,), jnp.float32),
        mesh=_mesh(),
        scratch_types=[
            pltpu.VMEM((STEPS, K), jnp.int32),
            pltpu.VMEM((K,), jnp.float32),
            pltpu.VMEM_SHARED((DEG_ACC,), jnp.float32),
            pltpu.SemaphoreType.DMA,
        ],
    )(dstp, zeros1)


CHUNKS = E_PAD // K
C0 = 512
Q0 = C0 // NS
Q1 = (CHUNKS - C0) // NS
QH = Q1 // 2


def _msg_body(g_hbm, src_hbm, dst_hbm, zero_hbm, out_hbm,
              srcv, dstv, bufA, bufB, sacc, gsem):
    c = lax.axis_index("c")
    s = lax.axis_index("s")
    pltpu.sync_copy(zero_hbm.at[pl.ds(s * MSG_SLAB, MSG_SLAB)],
                    sacc.at[pl.ds(s * MSG_SLAB, MSG_SLAB)])
    plsc.subcore_barrier()

    def fire_g(j, buf):
        pltpu.async_copy(g_hbm.at[srcv.at[j, pl.ds(0, KH)]],
                         buf.at[pl.ds(0, KH)], gsem)
        pltpu.async_copy(g_hbm.at[srcv.at[j, pl.ds(KH, KH)]],
                         buf.at[pl.ds(KH, KH)], gsem)

    def scat(j, buf):
        pltpu.sync_copy(buf, sacc.at[dstv.at[j]], add=True)

    def drain():
        pltpu.make_async_copy(g_hbm.at[pl.ds(0, KH)],
                              bufA.at[pl.ds(0, KH)], gsem).wait()
        pltpu.make_async_copy(g_hbm.at[pl.ds(0, KH)],
                              bufA.at[pl.ds(0, KH)], gsem).wait()

    def run(tile_base, qh, outer_n):
        for p in range(2):
            pltpu.sync_copy(src_hbm.at[pl.ds(tile_base + p * qh, qh)],
                            srcv.at[pl.ds(0, qh)])
            pltpu.sync_copy(dst_hbm.at[pl.ds(tile_base + p * qh, qh)],
                            dstv.at[pl.ds(0, qh)])
            fire_g(0, bufA)

            def outer(i, carry):
                a = 2 * i
                b_ = 2 * i + 1
                drain()
                fire_g(b_, bufB)
                scat(a, bufA)
                drain()

                @pl.when(i < outer_n - 1)
                def _():
                    fire_g(a + 2, bufA)

                scat(b_, bufB)
                return carry

            lax.fori_loop(0, outer_n, outer, 0)

    @pl.when(c == 0)
    def _():
        run(s * Q0, Q0 // 2, Q0 // 4)

    @pl.when(c == 1)
    def _():
        run(C0 + s * Q1, Q1 // 2, Q1 // 4)

    plsc.subcore_barrier()
    pltpu.sync_copy(sacc.at[pl.ds(s * MSG_SLAB, MSG_SLAB)],
                    out_hbm.at[c, pl.ds(s * MSG_SLAB, MSG_SLAB)])


def _msg_call(g, srcp, dstp, zeros2):
    return pl.kernel(
        _msg_body,
        out_type=jax.ShapeDtypeStruct((NC, MSG_ACC, D), jnp.float32),
        mesh=_mesh(),
        scratch_types=[
            pltpu.VMEM((QH, K), jnp.int32),
            pltpu.VMEM((QH, K), jnp.int32),
            pltpu.VMEM((K, D), jnp.float32),
            pltpu.VMEM((K, D), jnp.float32),
            pltpu.VMEM_SHARED((MSG_ACC, D), jnp.float32),
            pltpu.SemaphoreType.DMA,
        ],
    )(g, srcp, dstp, zeros2)


_MM_BM = 2000


def _mm_body(x_ref, w_ref, h_ref):
    h_ref[...] = jnp.dot(x_ref[...], w_ref[...],
                         preferred_element_type=jnp.float32)


def _mm_call(x, W):
    return pl.pallas_call(
        _mm_body,
        grid=(N // _MM_BM,),
        in_specs=[
            pl.BlockSpec((_MM_BM, D), lambda i: (i, 0)),
            pl.BlockSpec((D, D), lambda i: (0, 0)),
        ],
        out_specs=pl.BlockSpec((_MM_BM, D), lambda i: (i, 0)),
        out_shape=jax.ShapeDtypeStruct((N, D), jnp.float32),
    )(x, W)


def _scale_body(deg_ref, h_ref, g_ref):
    d = deg_ref[0] + deg_ref[1] + 1.0
    dinv = lax.rsqrt(d)
    g_ref[...] = h_ref[...] * dinv


def _scale_call(deg_col, h):
    bm = 2000
    return pl.pallas_call(
        _scale_body,
        grid=(N // bm,),
        in_specs=[
            pl.BlockSpec((NC, bm, 1), lambda i: (0, i, 0)),
            pl.BlockSpec((bm, D), lambda i: (i, 0)),
        ],
        out_specs=pl.BlockSpec((bm, D), lambda i: (i, 0)),
        out_shape=jax.ShapeDtypeStruct((N, D), jnp.float32),
    )(deg_col, h)


def _out_body(deg_ref, s_ref, g_ref, b_ref, o_ref):
    d = deg_ref[0] + deg_ref[1] + 1.0
    dinv = lax.rsqrt(d)
    o_ref[...] = dinv * (s_ref[0] + s_ref[1] + g_ref[...]) + b_ref[...]


def _out_call(deg_col, s2, g, b2):
    bm = 2000
    return pl.pallas_call(
        _out_body,
        grid=(N // bm,),
        in_specs=[
            pl.BlockSpec((NC, bm, 1), lambda i: (0, i, 0)),
            pl.BlockSpec((NC, bm, D), lambda i: (0, i, 0)),
            pl.BlockSpec((bm, D), lambda i: (i, 0)),
            pl.BlockSpec((1, D), lambda i: (0, 0)),
        ],
        out_specs=pl.BlockSpec((bm, D), lambda i: (i, 0)),
        out_shape=jax.ShapeDtypeStruct((N, D), jnp.float32),
    )(deg_col, s2, g, b2)


def kernel(x, edge_index, edge_attr, W, b):
    src = edge_index[0].astype(jnp.int32)
    dst = edge_index[1].astype(jnp.int32)
    pad = E_PAD - E
    srcp = jnp.concatenate(
        [src, jnp.zeros((pad,), jnp.int32)]).reshape(NW, STEPS, K)
    dstp = jnp.concatenate(
        [dst, jnp.full((pad,), DUMMY, jnp.int32)]).reshape(NW, STEPS, K)
    srcf = srcp.reshape(CHUNKS, K)
    dstf = dstp.reshape(CHUNKS, K)
    zeros1 = jnp.zeros((DEG_ACC,), jnp.float32)
    zeros2 = jnp.zeros((MSG_ACC, D), jnp.float32)

    deg2 = _deg_call(dstp, zeros1).reshape(NC, DEG_ACC)
    h = _mm_call(x, W)
    deg_col = deg2[:, :N].reshape(NC, N, 1)
    g = _scale_call(deg_col, h)
    s2 = _msg_call(g, srcf, dstf, zeros2)[:, :N, :]
    return _out_call(deg_col, s2, g, b.reshape(1, D))


# speedup vs baseline: 1.2579x; 1.0380x over previous
import functools

import jax
import jax.numpy as jnp
from jax import lax
from jax.experimental import pallas as pl
from jax.experimental.pallas import tpu as pltpu
from jax.experimental.pallas import tpu_sc as plsc

N = 10000
E = 320000
D = 128

NC = 2
NS = 16
NW = NC * NS

K = 128
KH = K // 2
STEPS = 80
HALF = STEPS // 2
E_PAD = NW * STEPS * K
DUMMY = N

DEG_ACC = 10240
DEG_SLAB = DEG_ACC // NS
MSG_ACC = 10112
MSG_SLAB = MSG_ACC // NS


def _mesh():
    return plsc.VectorSubcoreMesh(
        core_axis_name="c", subcore_axis_name="s", num_cores=NC, num_subcores=NS)


def _deg_body(dst_hbm, zero_hbm, out_hbm, dstv, onesv, dacc, sem):
    c = lax.axis_index("c")
    s = lax.axis_index("s")
    w = c * NS + s
    pltpu.sync_copy(dst_hbm.at[w], dstv)
    for i in range(K // 16):
        onesv[pl.ds(i * 16, 16)] = jnp.ones((16,), jnp.float32)
    pltpu.sync_copy(zero_hbm.at[pl.ds(s * DEG_SLAB, DEG_SLAB)],
                    dacc.at[pl.ds(s * DEG_SLAB, DEG_SLAB)])
    plsc.subcore_barrier()

    def step(j, carry):
        pltpu.sync_copy(onesv, dacc.at[dstv.at[j]], add=True)
        return carry

    lax.fori_loop(0, STEPS, step, 0)
    plsc.subcore_barrier()
    pltpu.sync_copy(dacc.at[pl.ds(s * DEG_SLAB, DEG_SLAB)],
                    out_hbm.at[pl.ds(c * DEG_ACC + s * DEG_SLAB, DEG_SLAB)])


def _deg_call(dstp, zeros1):
    return pl.kernel(
        _deg_body,
        out_type=jax.ShapeDtypeStruct((NC * DEG_ACC,), jnp.float32),
        mesh=_mesh(),
        scratch_types=[
            pltpu.VMEM((STEPS, K), jnp.int32),
            pltpu.VMEM((K,), jnp.float32),
            pltpu.VMEM_SHARED((DEG_ACC,), jnp.float32),
            pltpu.SemaphoreType.DMA,
        ],
    )(dstp, zeros1)


CHUNKS = E_PAD // K
C0 = 768
Q0 = C0 // NS
Q1 = (CHUNKS - C0) // NS
QH = 64


def _msg_body(g_hbm, src_hbm, dst_hbm, zero_hbm, out_hbm,
              srcv, dstv, bufA, bufB, sacc, gsem):
    c = lax.axis_index("c")
    s = lax.axis_index("s")
    pltpu.sync_copy(zero_hbm.at[pl.ds(s * MSG_SLAB, MSG_SLAB)],
                    sacc.at[pl.ds(s * MSG_SLAB, MSG_SLAB)])
    plsc.subcore_barrier()

    def fire_g(j, buf):
        pltpu.async_copy(g_hbm.at[srcv.at[j, pl.ds(0, KH)]],
                         buf.at[pl.ds(0, KH)], gsem)
        pltpu.async_copy(g_hbm.at[srcv.at[j, pl.ds(KH, KH)]],
                         buf.at[pl.ds(KH, KH)], gsem)

    def scat(j, buf):
        pltpu.sync_copy(buf, sacc.at[dstv.at[j]], add=True)

    def drain():
        pltpu.make_async_copy(g_hbm.at[pl.ds(0, KH)],
                              bufA.at[pl.ds(0, KH)], gsem).wait()
        pltpu.make_async_copy(g_hbm.at[pl.ds(0, KH)],
                              bufA.at[pl.ds(0, KH)], gsem).wait()

    def run(tile_base, qh, outer_n, phases):
        for p in range(phases):
            pltpu.sync_copy(src_hbm.at[pl.ds(tile_base + p * qh, qh)],
                            srcv.at[pl.ds(0, qh)])
            pltpu.sync_copy(dst_hbm.at[pl.ds(tile_base + p * qh, qh)],
                            dstv.at[pl.ds(0, qh)])
            fire_g(0, bufA)

            def outer(i, carry):
                a = 2 * i
                b_ = 2 * i + 1
                drain()
                fire_g(b_, bufB)
                scat(a, bufA)
                drain()

                @pl.when(i < outer_n - 1)
                def _():
                    fire_g(a + 2, bufA)

                scat(b_, bufB)
                return carry

            lax.fori_loop(0, outer_n, outer, 0)

    @pl.when(c == 0)
    def _():
        run(s * Q0, Q0, Q0 // 2, 1)

    @pl.when(c == 1)
    def _():
        run(C0 + s * Q1, Q1 // 2, Q1 // 4, 2)

    plsc.subcore_barrier()
    pltpu.sync_copy(sacc.at[pl.ds(s * MSG_SLAB, MSG_SLAB)],
                    out_hbm.at[c, pl.ds(s * MSG_SLAB, MSG_SLAB)])


def _msg_call(g, srcp, dstp, zeros2):
    return pl.kernel(
        _msg_body,
        out_type=jax.ShapeDtypeStruct((NC, MSG_ACC, D), jnp.float32),
        mesh=_mesh(),
        scratch_types=[
            pltpu.VMEM((QH, K), jnp.int32),
            pltpu.VMEM((QH, K), jnp.int32),
            pltpu.VMEM((K, D), jnp.float32),
            pltpu.VMEM((K, D), jnp.float32),
            pltpu.VMEM_SHARED((MSG_ACC, D), jnp.float32),
            pltpu.SemaphoreType.DMA,
        ],
    )(g, srcp, dstp, zeros2)


_MM_BM = 2000


def _mm_body(x_ref, w_ref, h_ref):
    h_ref[...] = jnp.dot(x_ref[...], w_ref[...],
                         preferred_element_type=jnp.float32)


def _mm_call(x, W):
    return pl.pallas_call(
        _mm_body,
        grid=(N // _MM_BM,),
        in_specs=[
            pl.BlockSpec((_MM_BM, D), lambda i: (i, 0)),
            pl.BlockSpec((D, D), lambda i: (0, 0)),
        ],
        out_specs=pl.BlockSpec((_MM_BM, D), lambda i: (i, 0)),
        out_shape=jax.ShapeDtypeStruct((N, D), jnp.float32),
    )(x, W)


def _scale_body(deg_ref, h_ref, g_ref):
    d = deg_ref[0] + deg_ref[1] + 1.0
    dinv = lax.rsqrt(d)
    g_ref[...] = h_ref[...] * dinv


def _scale_call(deg_col, h):
    bm = 2000
    return pl.pallas_call(
        _scale_body,
        grid=(N // bm,),
        in_specs=[
            pl.BlockSpec((NC, bm, 1), lambda i: (0, i, 0)),
            pl.BlockSpec((bm, D), lambda i: (i, 0)),
        ],
        out_specs=pl.BlockSpec((bm, D), lambda i: (i, 0)),
        out_shape=jax.ShapeDtypeStruct((N, D), jnp.float32),
    )(deg_col, h)


def _out_body(deg_ref, s_ref, g_ref, b_ref, o_ref):
    d = deg_ref[0] + deg_ref[1] + 1.0
    dinv = lax.rsqrt(d)
    o_ref[...] = dinv * (s_ref[0] + s_ref[1] + g_ref[...]) + b_ref[...]


def _out_call(deg_col, s2, g, b2):
    bm = 2000
    return pl.pallas_call(
        _out_body,
        grid=(N // bm,),
        in_specs=[
            pl.BlockSpec((NC, bm, 1), lambda i: (0, i, 0)),
            pl.BlockSpec((NC, bm, D), lambda i: (0, i, 0)),
            pl.BlockSpec((bm, D), lambda i: (i, 0)),
            pl.BlockSpec((1, D), lambda i: (0, 0)),
        ],
        out_specs=pl.BlockSpec((bm, D), lambda i: (i, 0)),
        out_shape=jax.ShapeDtypeStruct((N, D), jnp.float32),
    )(deg_col, s2, g, b2)


def kernel(x, edge_index, edge_attr, W, b):
    src = edge_index[0].astype(jnp.int32)
    dst = edge_index[1].astype(jnp.int32)
    pad = E_PAD - E
    srcp = jnp.concatenate(
        [src, jnp.zeros((pad,), jnp.int32)]).reshape(NW, STEPS, K)
    dstp = jnp.concatenate(
        [dst, jnp.full((pad,), DUMMY, jnp.int32)]).reshape(NW, STEPS, K)
    srcf = srcp.reshape(CHUNKS, K)
    dstf = dstp.reshape(CHUNKS, K)
    zeros1 = jnp.zeros((DEG_ACC,), jnp.float32)
    zeros2 = jnp.zeros((MSG_ACC, D), jnp.float32)

    deg2 = _deg_call(dstp, zeros1).reshape(NC, DEG_ACC)
    h = _mm_call(x, W)
    deg_col = deg2[:, :N].reshape(NC, N, 1)
    g = _scale_call(deg_col, h)
    s2 = _msg_call(g, srcf, dstf, zeros2)[:, :N, :]
    return _out_call(deg_col, s2, g, b.reshape(1, D))
